# Initial kernel scaffold; baseline (speedup 1.0000x reference)
import jax
import jax.numpy as jnp
from jax.experimental import pallas as pl


def kernel(z, v_in, g_in, b_in, codebook, v_out, g_out, b_out):
    raise NotImplementedError("write your pallas kernel here")


# speedup vs baseline: 1.3331x; 1.3331x over previous
import functools

import jax
import jax.numpy as jnp
from jax import lax
from jax.experimental import pallas as pl
from jax.experimental.pallas import tpu as pltpu
from jax.experimental.pallas import tpu_sc as plsc


def _prep_body(v_in_ref, g_in_ref, n_in_ref, v_out_ref, g_out_ref, n_out_ref,
               w_in_ref, w_out_ref):
    w_in_ref[...] = g_in_ref[...] * v_in_ref[...] / n_in_ref[...]
    w_out_ref[...] = g_out_ref[...] * v_out_ref[...] / n_out_ref[...]


def _vq_body(z_ref, w_in_ref, b_in_ref, cb_ref, cbn_ref, ze_ref, idx_ref,
             ze_bf_ref):
    z = z_ref[0]
    z_e = jnp.dot(w_in_ref[...], z, preferred_element_type=jnp.float32)
    z_e = z_e + b_in_ref[...]
    ze_ref[0] = z_e
    ze_bf_ref[...] = z_e.astype(jnp.bfloat16)
    dotp = lax.dot_general(cb_ref[...], ze_bf_ref[...], (((1,), (0,)), ((), ())),
                           preferred_element_type=jnp.float32)
    enorm = jnp.sum(z_e * z_e, axis=0, keepdims=True)
    neg = (2.0 * dotp - enorm) - cbn_ref[...]
    KC = 2048
    nck = neg.shape[0] // KC
    best = None
    bidx = None
    for j in range(nck):
        blk = neg[j * KC:(j + 1) * KC]
        cmax = jnp.max(blk, axis=0, keepdims=True)
        carg = (jnp.argmax(blk, axis=0, keepdims=True).astype(jnp.int32)
                + j * KC)
        if j == 0:
            best = cmax.astype(jnp.bfloat16)
            bidx = carg
        else:
            take = cmax > best.astype(jnp.float32)
            bidx = jnp.where(take, carg, bidx)
            best = jnp.where(take, cmax.astype(jnp.bfloat16), best)
    idx_ref[0, 0] = bidx[0]


def _out_body(zq_ref, w_out_ref, b_out_ref, out_ref):
    zq = zq_ref[...]
    w = w_out_ref[...]
    out = lax.dot_general(w, zq, (((1,), (1,)), ((), ())),
                          preferred_element_type=jnp.float32)
    out_ref[0] = out + b_out_ref[...]


def _make_sc_gather(K, C, NC, NS):
    NW = NC * NS
    rows_per_w = K // NW
    chunk = 128
    n_chunks = rows_per_w // chunk
    mesh = plsc.VectorSubcoreMesh(core_axis_name="c", subcore_axis_name="s")

    @functools.partial(
        pl.kernel,
        mesh=mesh,
        out_type=jax.ShapeDtypeStruct((K, C), jnp.float32),
        scratch_types=[
            pltpu.VMEM((n_chunks, chunk), jnp.int32),
            pltpu.VMEM((rows_per_w, C), jnp.float32),
            pltpu.SemaphoreType.DMA,
        ],
    )
    def gather_k(idx_hbm, table_hbm, out_hbm, idx_v, rows_v, sem):
        wid = lax.axis_index("s") * NC + lax.axis_index("c")
        base = wid * rows_per_w
        pltpu.sync_copy(idx_hbm.at[wid], idx_v)
        copies = []
        for j in range(n_chunks):
            copies.append(pltpu.async_copy(
                table_hbm.at[idx_v.at[j]],
                rows_v.at[pl.ds(j * chunk, chunk)], sem))
        for cp in copies:
            cp.wait()
        pltpu.sync_copy(rows_v, out_hbm.at[pl.ds(base, rows_per_w)])

    return gather_k


def kernel(z, v_in, g_in, b_in, codebook, v_out, g_out, b_out):
    B, D, T = z.shape
    C, _ = v_in.shape
    K, _ = codebook.shape
    TB = 256
    NT = T // TB

    z = z.astype(jnp.float32)
    cb = codebook.astype(jnp.float32)

    n_in = jnp.sqrt(jnp.sum(v_in * v_in, axis=1, keepdims=True))
    n_out = jnp.sqrt(jnp.sum(v_out * v_out, axis=1, keepdims=True))
    cbn = jnp.sum(cb ** 2, axis=1)[:, None]

    w_in, w_out = pl.pallas_call(
        _prep_body,
        out_shape=(
            jax.ShapeDtypeStruct((C, D), jnp.float32),
            jax.ShapeDtypeStruct((D, C), jnp.float32),
        ),
    )(v_in, g_in.reshape(C, 1), n_in, v_out, g_out.reshape(D, 1), n_out)

    ze, idx3 = pl.pallas_call(
        _vq_body,
        grid=(B, NT),
        in_specs=[
            pl.BlockSpec((1, D, TB), lambda b, t: (b, 0, t)),
            pl.BlockSpec((C, D), lambda b, t: (0, 0)),
            pl.BlockSpec((C, 1), lambda b, t: (0, 0)),
            pl.BlockSpec((K, C), lambda b, t: (0, 0)),
            pl.BlockSpec((K, 1), lambda b, t: (0, 0)),
        ],
        out_specs=(
            pl.BlockSpec((1, C, TB), lambda b, t: (b, 0, t)),
            pl.BlockSpec((1, 1, TB), lambda b, t: (b * NT + t, 0, 0)),
        ),
        out_shape=(
            jax.ShapeDtypeStruct((B, C, T), jnp.float32),
            jax.ShapeDtypeStruct((B * NT, 1, TB), jnp.int32),
        ),
        scratch_shapes=[pltpu.VMEM((C, TB), jnp.bfloat16)],
    )(z, w_in, b_in.reshape(C, 1), cb.astype(jnp.bfloat16), cbn)

    indices = idx3.reshape(B, T)

    info = plsc.get_sparse_core_info()
    NC, NS = info.num_cores, info.num_subcores
    NW = NC * NS
    gather_k = _make_sc_gather(K, C, NC, NS)
    zq = gather_k(indices.reshape(NW, -1, 128), cb)

    z_q_out = pl.pallas_call(
        _out_body,
        grid=(B, NT),
        in_specs=[
            pl.BlockSpec((TB, C), lambda b, t: (b * NT + t, 0)),
            pl.BlockSpec((D, C), lambda b, t: (0, 0)),
            pl.BlockSpec((D, 1), lambda b, t: (0, 0)),
        ],
        out_specs=pl.BlockSpec((1, D, TB), lambda b, t: (b, 0, t)),
        out_shape=jax.ShapeDtypeStruct((B, D, T), jnp.float32),
    )(zq, w_out, b_out.reshape(D, 1))

    return (z_q_out, indices, ze)
